# baseline (speedup 1.0000x reference)
import functools

import jax
import jax.numpy as jnp
from jax.experimental import pallas as pl
from jax.experimental.pallas import tpu as pltpu
from jax.experimental.pallas import tpu_sc as plsc

H = 96
W = 96
N = H * W
NPAD = N + 256
CH = 64
CHP = 128
BR = 256
CT = 512
NEG = -1e30


def _attn_kernel(pq_ref, f_blk, lq_ref, colneg_ref, invn_ref, out_blk):
    i = pl.program_id(0)
    p_cnt = pq_ref[0]
    q_cnt = pq_ref[1]

    @pl.when(i * BR < p_cnt)
    def _compute():
        f = f_blk[...][:, :CH]
        fh = f.astype(jnp.bfloat16)
        fl = (f - fh.astype(jnp.float32)).astype(jnp.bfloat16)
        tj = (q_cnt + CT - 1) // CT

        def body(j, carry):
            s, acc = carry
            lt = lq_ref[pl.ds(j * CT, CT), :][:, :CH]
            lth = lt.astype(jnp.bfloat16)
            ltl = (lt - lth.astype(jnp.float32)).astype(jnp.bfloat16)
            nt = (((1,), (1,)), ((), ()))
            logits = (
                jax.lax.dot_general(
                    fh, lth, nt, preferred_element_type=jnp.float32)
                + (jax.lax.dot_general(
                    fh, ltl, nt, preferred_element_type=jnp.float32)
                   + jax.lax.dot_general(
                    fl, lth, nt, preferred_element_type=jnp.float32)))
            p = jnp.exp(logits * invn_ref[:, pl.ds(j * CT, CT)]
                        + colneg_ref[:, pl.ds(j * CT, CT)])
            s_new = s + jnp.sum(p, axis=1, keepdims=True)
            acc_new = acc + jax.lax.dot_general(
                p.astype(jnp.bfloat16), lth, (((1,), (0,)), ((), ())),
                preferred_element_type=jnp.float32)
            return s_new, acc_new

        s0 = jnp.zeros((BR, 1), jnp.float32)
        a0 = jnp.zeros((BR, CH), jnp.float32)
        s, acc = jax.lax.fori_loop(0, tj, body, (s0, a0))
        out_blk[...] = jnp.concatenate(
            [acc / s, jnp.zeros((BR, CHP - CH), jnp.float32)], axis=1)

    @pl.when(i * BR >= p_cnt)
    def _zero():
        out_blk[...] = jnp.zeros((BR, CHP), jnp.float32)


TB = 512


def _pre_kernel(x_blk, f_out, l_out):
    xt = jnp.transpose(x_blk[...])
    z = jnp.zeros((TB, CHP - CH), jnp.float32)
    f_out[...] = jnp.concatenate([xt[:, :CH], z], axis=1)
    l_out[...] = jnp.concatenate([xt[:, CH:], z], axis=1)


def _tc_pre(x):
    return pl.pallas_call(
        _pre_kernel,
        grid=(N // TB,),
        in_specs=[pl.BlockSpec((CHP, TB), lambda i: (0, i))],
        out_specs=[pl.BlockSpec((TB, CHP), lambda i: (i, 0)),
                   pl.BlockSpec((TB, CHP), lambda i: (i, 0))],
        out_shape=[jax.ShapeDtypeStruct((N, CHP), jnp.float32),
                   jax.ShapeDtypeStruct((N, CHP), jnp.float32)],
        compiler_params=pltpu.CompilerParams(
            dimension_semantics=("arbitrary",)),
    )(x)


def _post_kernel(x_blk, out_blk):
    out_blk[...] = jnp.transpose(x_blk[...][:, :CH])


def _tc_post(x):
    return pl.pallas_call(
        _post_kernel,
        grid=(N // TB,),
        in_specs=[pl.BlockSpec((TB, CHP), lambda i: (i, 0))],
        out_specs=pl.BlockSpec((CH, TB), lambda i: (0, i)),
        out_shape=jax.ShapeDtypeStruct((CH, N), jnp.float32),
        compiler_params=pltpu.CompilerParams(
            dimension_semantics=("arbitrary",)),
    )(x)


NC = 2
NS = 16
NWORK = NC * NS
S = N // NWORK
JCH = 3
SC = S // JCH

@functools.cache
def _sc_mesh():
    return plsc.VectorSubcoreMesh(core_axis_name="c", subcore_axis_name="s")


def _compact_body(f2d, l2d, srcf, srcl, fp, lq, idxf_v, idxl_v, rows_f, rows_l, sem):
    wid = jax.lax.axis_index("s") * NC + jax.lax.axis_index("c")
    base = wid * S
    pltpu.sync_copy(srcf.at[wid], idxf_v)
    pltpu.sync_copy(srcl.at[wid], idxl_v)
    copies = []
    for j in range(JCH):
        copies.append(pltpu.async_copy(
            f2d.at[idxf_v.at[j]], rows_f.at[pl.ds(j * SC, SC)], sem))
        copies.append(pltpu.async_copy(
            l2d.at[idxl_v.at[j]], rows_l.at[pl.ds(j * SC, SC)], sem))
    for cp in copies:
        cp.wait()
    pltpu.sync_copy(rows_f, fp.at[pl.ds(base, S)])
    pltpu.sync_copy(rows_l, lq.at[pl.ds(base, S)])


def _sc_compact(f2d, l2d, srcf, srcl):
    return pl.kernel(
        _compact_body,
        out_type=(jax.ShapeDtypeStruct((NPAD, CHP), jnp.float32),
                  jax.ShapeDtypeStruct((NPAD, CHP), jnp.float32)),
        mesh=_sc_mesh(),
        scratch_types=[
            pltpu.VMEM((JCH, SC), jnp.int32),
            pltpu.VMEM((JCH, SC), jnp.int32),
            pltpu.VMEM((S, CHP), jnp.float32),
            pltpu.VMEM((S, CHP), jnp.float32),
            pltpu.SemaphoreType.DMA,
        ],
    )(f2d, l2d, srcf, srcl)


def _paste_body(shifted, posx, out, idx_v, rows_v, sem):
    wid = jax.lax.axis_index("s") * NC + jax.lax.axis_index("c")
    base = wid * S
    pltpu.sync_copy(posx.at[wid], idx_v)
    copies = [pltpu.async_copy(shifted.at[idx_v.at[j]],
                               rows_v.at[pl.ds(j * SC, SC)], sem)
              for j in range(JCH)]
    for cp in copies:
        cp.wait()
    pltpu.sync_copy(rows_v, out.at[pl.ds(base, S)])


def _sc_paste(shifted, posx):
    return pl.kernel(
        _paste_body,
        out_type=jax.ShapeDtypeStruct((N, CHP), jnp.float32),
        mesh=_sc_mesh(),
        scratch_types=[
            pltpu.VMEM((JCH, SC), jnp.int32),
            pltpu.VMEM((S, CHP), jnp.float32),
            pltpu.SemaphoreType.DMA,
        ],
    )(shifted, posx)


def kernel(input, mask, shift_sz, stride, triple_w, flag):
    bz, c, h, w = input.shape
    ch = c // 2
    f2d, l2d = _tc_pre(input[0].reshape(c, N))
    flag = flag.astype(jnp.int32)
    is_m = flag == 1
    mcum = jnp.cumsum(is_m.astype(jnp.int32))
    ucum = jnp.cumsum(1 - is_m.astype(jnp.int32))
    p_cnt = mcum[-1]
    q_cnt = N - p_cnt

    iota = jnp.arange(N, dtype=jnp.int32)
    srcf = jnp.argsort(jnp.where(is_m, iota, iota + N)).astype(
        jnp.int32).reshape(NWORK, JCH, SC)
    srcl = jnp.argsort(jnp.where(is_m, iota + N, iota)).astype(
        jnp.int32).reshape(NWORK, JCH, SC)
    fp, lq = _sc_compact(f2d, l2d, srcf, srcl)
    live = jnp.arange(NPAD, dtype=jnp.int32) < q_cnt
    colneg = jnp.where(live, 0.0, NEG).astype(jnp.float32).reshape(1, NPAD)
    invn = jnp.where(live, jax.lax.rsqrt(jnp.sum(lq * lq, axis=1)),
                     0.0).astype(jnp.float32).reshape(1, NPAD)
    pq = jnp.stack([p_cnt, q_cnt]).astype(jnp.int32)

    grid_spec = pltpu.PrefetchScalarGridSpec(
        num_scalar_prefetch=1,
        grid=(NPAD // BR,),
        in_specs=[
            pl.BlockSpec((BR, CHP), lambda i, pq: (i, 0)),
            pl.BlockSpec((NPAD, CHP), lambda i, pq: (0, 0)),
            pl.BlockSpec((1, NPAD), lambda i, pq: (0, 0)),
            pl.BlockSpec((1, NPAD), lambda i, pq: (0, 0)),
        ],
        out_specs=pl.BlockSpec((BR, CHP), lambda i, pq: (i, 0)),
    )
    shifted = pl.pallas_call(
        _attn_kernel,
        grid_spec=grid_spec,
        out_shape=jax.ShapeDtypeStruct((NPAD, CHP), jnp.float32),
        compiler_params=pltpu.CompilerParams(
            dimension_semantics=("arbitrary",)),
    )(pq, fp, lq, colneg, invn)

    posx = jnp.where(is_m, mcum - 1, N + (iota % (NPAD - N))).reshape(
        NWORK, JCH, SC)
    out2d = _sc_paste(shifted, posx)
    former_masked = _tc_post(out2d).reshape(1, ch, h, w)
    return jnp.concatenate([input, former_masked], axis=1)


# speedup vs baseline: 1.3753x; 1.3753x over previous
import functools

import jax
import jax.numpy as jnp
from jax.experimental import pallas as pl
from jax.experimental.pallas import tpu as pltpu
from jax.experimental.pallas import tpu_sc as plsc

H = 96
W = 96
N = H * W
NPAD = N + 256
CH = 64
CHP = 128
BR = 256
CT = 1024
NEG = -1e30


def _attn_kernel(pq_ref, f_blk, lq_ref, colneg_ref, invn_ref, out_blk):
    i = pl.program_id(0)
    p_cnt = pq_ref[0]
    q_cnt = pq_ref[1]

    @pl.when(i * BR < p_cnt)
    def _compute():
        f = f_blk[...][:, :CH]
        tj = (q_cnt + CT - 1) // CT

        def body(j, carry):
            s, acc = carry
            lt = lq_ref[pl.ds(j * CT, CT), :][:, :CH]
            logits = jax.lax.dot_general(
                f, lt, (((1,), (1,)), ((), ())),
                preferred_element_type=jnp.float32)
            p = jnp.exp(logits * invn_ref[:, pl.ds(j * CT, CT)]
                        + colneg_ref[:, pl.ds(j * CT, CT)])
            s_new = s + jnp.sum(p, axis=1, keepdims=True)
            acc_new = acc + jax.lax.dot_general(
                p, lt, (((1,), (0,)), ((), ())),
                preferred_element_type=jnp.float32)
            return s_new, acc_new

        s0 = jnp.zeros((BR, 1), jnp.float32)
        a0 = jnp.zeros((BR, CH), jnp.float32)
        s, acc = jax.lax.fori_loop(0, tj, body, (s0, a0))
        out_blk[...] = jnp.concatenate(
            [acc / s, jnp.zeros((BR, CHP - CH), jnp.float32)], axis=1)

    @pl.when(i * BR >= p_cnt)
    def _zero():
        out_blk[...] = jnp.zeros((BR, CHP), jnp.float32)


TB = 512


def _pre_kernel(x_blk, f_out, l_out):
    xt = jnp.transpose(x_blk[...])
    z = jnp.zeros((TB, CHP - CH), jnp.float32)
    f_out[...] = jnp.concatenate([xt[:, :CH], z], axis=1)
    l_out[...] = jnp.concatenate([xt[:, CH:], z], axis=1)


def _tc_pre(x):
    return pl.pallas_call(
        _pre_kernel,
        grid=(N // TB,),
        in_specs=[pl.BlockSpec((CHP, TB), lambda i: (0, i))],
        out_specs=[pl.BlockSpec((TB, CHP), lambda i: (i, 0)),
                   pl.BlockSpec((TB, CHP), lambda i: (i, 0))],
        out_shape=[jax.ShapeDtypeStruct((N, CHP), jnp.float32),
                   jax.ShapeDtypeStruct((N, CHP), jnp.float32)],
        compiler_params=pltpu.CompilerParams(
            dimension_semantics=("arbitrary",)),
    )(x)


def _post_kernel(x_blk, out_blk):
    out_blk[...] = jnp.transpose(x_blk[...][:, :CH])


def _tc_post(x):
    return pl.pallas_call(
        _post_kernel,
        grid=(N // TB,),
        in_specs=[pl.BlockSpec((TB, CHP), lambda i: (i, 0))],
        out_specs=pl.BlockSpec((CH, TB), lambda i: (0, i)),
        out_shape=jax.ShapeDtypeStruct((CH, N), jnp.float32),
        compiler_params=pltpu.CompilerParams(
            dimension_semantics=("arbitrary",)),
    )(x)


NC = 2
NS = 16
NWORK = NC * NS
S = N // NWORK
JCH = 3
SC = S // JCH

@functools.cache
def _sc_mesh():
    return plsc.VectorSubcoreMesh(core_axis_name="c", subcore_axis_name="s")


def _compact_body(f2d, l2d, srcf, srcl, fp, lq, idxf_v, idxl_v, rows_f, rows_l, sem):
    wid = jax.lax.axis_index("s") * NC + jax.lax.axis_index("c")
    base = wid * S
    pltpu.sync_copy(srcf.at[wid], idxf_v)
    pltpu.sync_copy(srcl.at[wid], idxl_v)
    copies = []
    for j in range(JCH):
        copies.append(pltpu.async_copy(
            f2d.at[idxf_v.at[j]], rows_f.at[pl.ds(j * SC, SC)], sem))
        copies.append(pltpu.async_copy(
            l2d.at[idxl_v.at[j]], rows_l.at[pl.ds(j * SC, SC)], sem))
    for cp in copies:
        cp.wait()
    pltpu.sync_copy(rows_f, fp.at[pl.ds(base, S)])
    pltpu.sync_copy(rows_l, lq.at[pl.ds(base, S)])


def _sc_compact(f2d, l2d, srcf, srcl):
    return pl.kernel(
        _compact_body,
        out_type=(jax.ShapeDtypeStruct((NPAD, CHP), jnp.float32),
                  jax.ShapeDtypeStruct((NPAD, CHP), jnp.float32)),
        mesh=_sc_mesh(),
        scratch_types=[
            pltpu.VMEM((JCH, SC), jnp.int32),
            pltpu.VMEM((JCH, SC), jnp.int32),
            pltpu.VMEM((S, CHP), jnp.float32),
            pltpu.VMEM((S, CHP), jnp.float32),
            pltpu.SemaphoreType.DMA,
        ],
    )(f2d, l2d, srcf, srcl)


def _paste_body(shifted, posx, out, idx_v, rows_v, sem):
    wid = jax.lax.axis_index("s") * NC + jax.lax.axis_index("c")
    base = wid * S
    pltpu.sync_copy(posx.at[wid], idx_v)
    copies = [pltpu.async_copy(shifted.at[idx_v.at[j]],
                               rows_v.at[pl.ds(j * SC, SC)], sem)
              for j in range(JCH)]
    for cp in copies:
        cp.wait()
    pltpu.sync_copy(rows_v, out.at[pl.ds(base, S)])


def _sc_paste(shifted, posx):
    return pl.kernel(
        _paste_body,
        out_type=jax.ShapeDtypeStruct((N, CHP), jnp.float32),
        mesh=_sc_mesh(),
        scratch_types=[
            pltpu.VMEM((JCH, SC), jnp.int32),
            pltpu.VMEM((S, CHP), jnp.float32),
            pltpu.SemaphoreType.DMA,
        ],
    )(shifted, posx)


def kernel(input, mask, shift_sz, stride, triple_w, flag):
    bz, c, h, w = input.shape
    ch = c // 2
    f2d, l2d = _tc_pre(input[0].reshape(c, N))
    flag = flag.astype(jnp.int32)
    is_m = flag == 1
    mcum = jnp.cumsum(is_m.astype(jnp.int32))
    ucum = jnp.cumsum(1 - is_m.astype(jnp.int32))
    p_cnt = mcum[-1]
    q_cnt = N - p_cnt

    iota = jnp.arange(N, dtype=jnp.int32)
    srcf = jnp.argsort(jnp.where(is_m, iota, iota + N)).astype(
        jnp.int32).reshape(NWORK, JCH, SC)
    srcl = jnp.argsort(jnp.where(is_m, iota + N, iota)).astype(
        jnp.int32).reshape(NWORK, JCH, SC)
    fp, lq = _sc_compact(f2d, l2d, srcf, srcl)
    live = jnp.arange(NPAD, dtype=jnp.int32) < q_cnt
    colneg = jnp.where(live, 0.0, NEG).astype(jnp.float32).reshape(1, NPAD)
    invn = jnp.where(live, jax.lax.rsqrt(jnp.sum(lq * lq, axis=1)),
                     0.0).astype(jnp.float32).reshape(1, NPAD)
    pq = jnp.stack([p_cnt, q_cnt]).astype(jnp.int32)

    grid_spec = pltpu.PrefetchScalarGridSpec(
        num_scalar_prefetch=1,
        grid=(NPAD // BR,),
        in_specs=[
            pl.BlockSpec((BR, CHP), lambda i, pq: (i, 0)),
            pl.BlockSpec((NPAD, CHP), lambda i, pq: (0, 0)),
            pl.BlockSpec((1, NPAD), lambda i, pq: (0, 0)),
            pl.BlockSpec((1, NPAD), lambda i, pq: (0, 0)),
        ],
        out_specs=pl.BlockSpec((BR, CHP), lambda i, pq: (i, 0)),
    )
    shifted = pl.pallas_call(
        _attn_kernel,
        grid_spec=grid_spec,
        out_shape=jax.ShapeDtypeStruct((NPAD, CHP), jnp.float32),
        compiler_params=pltpu.CompilerParams(
            dimension_semantics=("arbitrary",)),
    )(pq, fp, lq, colneg, invn)

    posx = jnp.where(is_m, mcum - 1, N + (iota % (NPAD - N))).reshape(
        NWORK, JCH, SC)
    out2d = _sc_paste(shifted, posx)
    former_masked = _tc_post(out2d).reshape(1, ch, h, w)
    return jnp.concatenate([input, former_masked], axis=1)


# speedup vs baseline: 1.4784x; 1.0749x over previous
import functools

import jax
import jax.numpy as jnp
from jax.experimental import pallas as pl
from jax.experimental.pallas import tpu as pltpu
from jax.experimental.pallas import tpu_sc as plsc

H = 96
W = 96
N = H * W
NPAD = N + 256
CH = 64
CHP = 128
BR = 512
CT = 1024
NEG = -1e30


def _attn_kernel(pq_ref, f_blk, lq_ref, colneg_ref, invn_ref, out_blk):
    i = pl.program_id(0)
    p_cnt = pq_ref[0]
    q_cnt = pq_ref[1]

    @pl.when(i * BR < p_cnt)
    def _compute():
        f = f_blk[...][:, :CH]
        tj = (q_cnt + CT - 1) // CT

        def body(j, carry):
            s, acc = carry
            lt = lq_ref[pl.ds(j * CT, CT), :][:, :CH]
            logits = jax.lax.dot_general(
                f, lt, (((1,), (1,)), ((), ())),
                preferred_element_type=jnp.float32)
            p = jnp.exp(logits * invn_ref[:, pl.ds(j * CT, CT)]
                        + colneg_ref[:, pl.ds(j * CT, CT)])
            s_new = s + jnp.sum(p, axis=1, keepdims=True)
            acc_new = acc + jax.lax.dot_general(
                p, lt, (((1,), (0,)), ((), ())),
                preferred_element_type=jnp.float32)
            return s_new, acc_new

        s0 = jnp.zeros((BR, 1), jnp.float32)
        a0 = jnp.zeros((BR, CH), jnp.float32)
        s, acc = jax.lax.fori_loop(0, tj, body, (s0, a0))
        out_blk[...] = jnp.concatenate(
            [acc / s, jnp.zeros((BR, CHP - CH), jnp.float32)], axis=1)

    @pl.when(i * BR >= p_cnt)
    def _zero():
        out_blk[...] = jnp.zeros((BR, CHP), jnp.float32)


TB = 512


def _pre_kernel(x_blk, f_out, l_out):
    xt = jnp.transpose(x_blk[...])
    z = jnp.zeros((TB, CHP - CH), jnp.float32)
    f_out[...] = jnp.concatenate([xt[:, :CH], z], axis=1)
    l_out[...] = jnp.concatenate([xt[:, CH:], z], axis=1)


def _tc_pre(x):
    return pl.pallas_call(
        _pre_kernel,
        grid=(N // TB,),
        in_specs=[pl.BlockSpec((CHP, TB), lambda i: (0, i))],
        out_specs=[pl.BlockSpec((TB, CHP), lambda i: (i, 0)),
                   pl.BlockSpec((TB, CHP), lambda i: (i, 0))],
        out_shape=[jax.ShapeDtypeStruct((N, CHP), jnp.float32),
                   jax.ShapeDtypeStruct((N, CHP), jnp.float32)],
        compiler_params=pltpu.CompilerParams(
            dimension_semantics=("arbitrary",)),
    )(x)


def _post_kernel(x_blk, out_blk):
    out_blk[...] = jnp.transpose(x_blk[...][:, :CH])


def _tc_post(x):
    return pl.pallas_call(
        _post_kernel,
        grid=(N // TB,),
        in_specs=[pl.BlockSpec((TB, CHP), lambda i: (i, 0))],
        out_specs=pl.BlockSpec((CH, TB), lambda i: (0, i)),
        out_shape=jax.ShapeDtypeStruct((CH, N), jnp.float32),
        compiler_params=pltpu.CompilerParams(
            dimension_semantics=("arbitrary",)),
    )(x)


NC = 2
NS = 16
NWORK = NC * NS
S = N // NWORK
JCH = 3
SC = S // JCH

@functools.cache
def _sc_mesh():
    return plsc.VectorSubcoreMesh(core_axis_name="c", subcore_axis_name="s")


def _compact_body(f2d, l2d, srcf, srcl, fp, lq, idxf_v, idxl_v, rows_f, rows_l, sem):
    wid = jax.lax.axis_index("s") * NC + jax.lax.axis_index("c")
    base = wid * S
    pltpu.sync_copy(srcf.at[wid], idxf_v)
    pltpu.sync_copy(srcl.at[wid], idxl_v)
    copies = []
    for j in range(JCH):
        copies.append(pltpu.async_copy(
            f2d.at[idxf_v.at[j]], rows_f.at[pl.ds(j * SC, SC)], sem))
        copies.append(pltpu.async_copy(
            l2d.at[idxl_v.at[j]], rows_l.at[pl.ds(j * SC, SC)], sem))
    for cp in copies:
        cp.wait()
    pltpu.sync_copy(rows_f, fp.at[pl.ds(base, S)])
    pltpu.sync_copy(rows_l, lq.at[pl.ds(base, S)])


def _sc_compact(f2d, l2d, srcf, srcl):
    return pl.kernel(
        _compact_body,
        out_type=(jax.ShapeDtypeStruct((NPAD, CHP), jnp.float32),
                  jax.ShapeDtypeStruct((NPAD, CHP), jnp.float32)),
        mesh=_sc_mesh(),
        scratch_types=[
            pltpu.VMEM((JCH, SC), jnp.int32),
            pltpu.VMEM((JCH, SC), jnp.int32),
            pltpu.VMEM((S, CHP), jnp.float32),
            pltpu.VMEM((S, CHP), jnp.float32),
            pltpu.SemaphoreType.DMA,
        ],
    )(f2d, l2d, srcf, srcl)


def _paste_body(shifted, posx, out, idx_v, rows_v, sem):
    wid = jax.lax.axis_index("s") * NC + jax.lax.axis_index("c")
    base = wid * S
    pltpu.sync_copy(posx.at[wid], idx_v)
    copies = [pltpu.async_copy(shifted.at[idx_v.at[j]],
                               rows_v.at[pl.ds(j * SC, SC)], sem)
              for j in range(JCH)]
    for cp in copies:
        cp.wait()
    pltpu.sync_copy(rows_v, out.at[pl.ds(base, S)])


def _sc_paste(shifted, posx):
    return pl.kernel(
        _paste_body,
        out_type=jax.ShapeDtypeStruct((N, CHP), jnp.float32),
        mesh=_sc_mesh(),
        scratch_types=[
            pltpu.VMEM((JCH, SC), jnp.int32),
            pltpu.VMEM((S, CHP), jnp.float32),
            pltpu.SemaphoreType.DMA,
        ],
    )(shifted, posx)


def kernel(input, mask, shift_sz, stride, triple_w, flag):
    bz, c, h, w = input.shape
    ch = c // 2
    f2d, l2d = _tc_pre(input[0].reshape(c, N))
    flag = flag.astype(jnp.int32)
    is_m = flag == 1
    mcum = jnp.cumsum(is_m.astype(jnp.int32))
    ucum = jnp.cumsum(1 - is_m.astype(jnp.int32))
    p_cnt = mcum[-1]
    q_cnt = N - p_cnt

    iota = jnp.arange(N, dtype=jnp.int32)
    srcf = jnp.argsort(jnp.where(is_m, iota, iota + N)).astype(
        jnp.int32).reshape(NWORK, JCH, SC)
    srcl = jnp.argsort(jnp.where(is_m, iota + N, iota)).astype(
        jnp.int32).reshape(NWORK, JCH, SC)
    fp, lq = _sc_compact(f2d, l2d, srcf, srcl)
    live = jnp.arange(NPAD, dtype=jnp.int32) < q_cnt
    colneg = jnp.where(live, 0.0, NEG).astype(jnp.float32).reshape(1, NPAD)
    invn = jnp.where(live, jax.lax.rsqrt(jnp.sum(lq * lq, axis=1)),
                     0.0).astype(jnp.float32).reshape(1, NPAD)
    pq = jnp.stack([p_cnt, q_cnt]).astype(jnp.int32)

    grid_spec = pltpu.PrefetchScalarGridSpec(
        num_scalar_prefetch=1,
        grid=(NPAD // BR,),
        in_specs=[
            pl.BlockSpec((BR, CHP), lambda i, pq: (i, 0)),
            pl.BlockSpec((NPAD, CHP), lambda i, pq: (0, 0)),
            pl.BlockSpec((1, NPAD), lambda i, pq: (0, 0)),
            pl.BlockSpec((1, NPAD), lambda i, pq: (0, 0)),
        ],
        out_specs=pl.BlockSpec((BR, CHP), lambda i, pq: (i, 0)),
    )
    shifted = pl.pallas_call(
        _attn_kernel,
        grid_spec=grid_spec,
        out_shape=jax.ShapeDtypeStruct((NPAD, CHP), jnp.float32),
        compiler_params=pltpu.CompilerParams(
            dimension_semantics=("arbitrary",)),
    )(pq, fp, lq, colneg, invn)

    posx = jnp.where(is_m, mcum - 1, N + (iota % (NPAD - N))).reshape(
        NWORK, JCH, SC)
    out2d = _sc_paste(shifted, posx)
    former_masked = _tc_post(out2d).reshape(1, ch, h, w)
    return jnp.concatenate([input, former_masked], axis=1)


# speedup vs baseline: 1.4799x; 1.0010x over previous
import functools

import jax
import jax.numpy as jnp
from jax.experimental import pallas as pl
from jax.experimental.pallas import tpu as pltpu
from jax.experimental.pallas import tpu_sc as plsc

H = 96
W = 96
N = H * W
NPAD = N + 512
CH = 64
CHP = 128
BR = 512
CT = 1024
NEG = -1e30


def _attn_kernel(pq_ref, f_blk, lq_ref, colneg_ref, invn_ref, out_blk):
    i = pl.program_id(0)
    p_cnt = pq_ref[0]
    q_cnt = pq_ref[1]

    @pl.when(i * BR < p_cnt)
    def _compute():
        f = f_blk[...][:, :CH]
        tj = (q_cnt + CT - 1) // CT

        def body(j, carry):
            s, acc = carry
            lt = lq_ref[pl.ds(j * CT, CT), :][:, :CH]
            logits = jax.lax.dot_general(
                f, lt, (((1,), (1,)), ((), ())),
                preferred_element_type=jnp.float32)
            p = jnp.exp(logits * invn_ref[:, pl.ds(j * CT, CT)]
                        + colneg_ref[:, pl.ds(j * CT, CT)])
            s_new = s + jnp.sum(p, axis=1, keepdims=True)
            acc_new = acc + jax.lax.dot_general(
                p, lt, (((1,), (0,)), ((), ())),
                preferred_element_type=jnp.float32)
            return s_new, acc_new

        s0 = jnp.zeros((BR, 1), jnp.float32)
        a0 = jnp.zeros((BR, CH), jnp.float32)
        s, acc = jax.lax.fori_loop(0, tj, body, (s0, a0))
        out_blk[...] = jnp.concatenate(
            [acc / s, jnp.zeros((BR, CHP - CH), jnp.float32)], axis=1)

    @pl.when(i * BR >= p_cnt)
    def _zero():
        out_blk[...] = jnp.zeros((BR, CHP), jnp.float32)


TB = 512


def _pre_kernel(x_blk, f_out, l_out):
    xt = jnp.transpose(x_blk[...])
    z = jnp.zeros((TB, CHP - CH), jnp.float32)
    f_out[...] = jnp.concatenate([xt[:, :CH], z], axis=1)
    l_out[...] = jnp.concatenate([xt[:, CH:], z], axis=1)


def _tc_pre(x):
    return pl.pallas_call(
        _pre_kernel,
        grid=(N // TB,),
        in_specs=[pl.BlockSpec((CHP, TB), lambda i: (0, i))],
        out_specs=[pl.BlockSpec((TB, CHP), lambda i: (i, 0)),
                   pl.BlockSpec((TB, CHP), lambda i: (i, 0))],
        out_shape=[jax.ShapeDtypeStruct((N, CHP), jnp.float32),
                   jax.ShapeDtypeStruct((N, CHP), jnp.float32)],
        compiler_params=pltpu.CompilerParams(
            dimension_semantics=("arbitrary",)),
    )(x)


def _post_kernel(x_blk, out_blk):
    out_blk[...] = jnp.transpose(x_blk[...][:, :CH])


def _tc_post(x):
    return pl.pallas_call(
        _post_kernel,
        grid=(N // TB,),
        in_specs=[pl.BlockSpec((TB, CHP), lambda i: (i, 0))],
        out_specs=pl.BlockSpec((CH, TB), lambda i: (0, i)),
        out_shape=jax.ShapeDtypeStruct((CH, N), jnp.float32),
        compiler_params=pltpu.CompilerParams(
            dimension_semantics=("arbitrary",)),
    )(x)


NC = 2
NS = 16
NWORK = NC * NS
S = N // NWORK
JCH = 3
SC = S // JCH

@functools.cache
def _sc_mesh():
    return plsc.VectorSubcoreMesh(core_axis_name="c", subcore_axis_name="s")


def _compact_body(f2d, l2d, srcf, srcl, fp, lq, idxf_v, idxl_v, rows_f, rows_l, sem):
    wid = jax.lax.axis_index("s") * NC + jax.lax.axis_index("c")
    base = wid * S
    pltpu.sync_copy(srcf.at[wid], idxf_v)
    pltpu.sync_copy(srcl.at[wid], idxl_v)
    copies = []
    for j in range(JCH):
        copies.append(pltpu.async_copy(
            f2d.at[idxf_v.at[j]], rows_f.at[pl.ds(j * SC, SC)], sem))
        copies.append(pltpu.async_copy(
            l2d.at[idxl_v.at[j]], rows_l.at[pl.ds(j * SC, SC)], sem))
    for cp in copies:
        cp.wait()
    pltpu.sync_copy(rows_f, fp.at[pl.ds(base, S)])
    pltpu.sync_copy(rows_l, lq.at[pl.ds(base, S)])


def _sc_compact(f2d, l2d, srcf, srcl):
    return pl.kernel(
        _compact_body,
        out_type=(jax.ShapeDtypeStruct((NPAD, CHP), jnp.float32),
                  jax.ShapeDtypeStruct((NPAD, CHP), jnp.float32)),
        mesh=_sc_mesh(),
        scratch_types=[
            pltpu.VMEM((JCH, SC), jnp.int32),
            pltpu.VMEM((JCH, SC), jnp.int32),
            pltpu.VMEM((S, CHP), jnp.float32),
            pltpu.VMEM((S, CHP), jnp.float32),
            pltpu.SemaphoreType.DMA,
        ],
    )(f2d, l2d, srcf, srcl)


def _paste_body(shifted, posx, out, idx_v, rows_v, sem):
    wid = jax.lax.axis_index("s") * NC + jax.lax.axis_index("c")
    base = wid * S
    pltpu.sync_copy(posx.at[wid], idx_v)
    copies = [pltpu.async_copy(shifted.at[idx_v.at[j]],
                               rows_v.at[pl.ds(j * SC, SC)], sem)
              for j in range(JCH)]
    for cp in copies:
        cp.wait()
    pltpu.sync_copy(rows_v, out.at[pl.ds(base, S)])


def _sc_paste(shifted, posx):
    return pl.kernel(
        _paste_body,
        out_type=jax.ShapeDtypeStruct((N, CHP), jnp.float32),
        mesh=_sc_mesh(),
        scratch_types=[
            pltpu.VMEM((JCH, SC), jnp.int32),
            pltpu.VMEM((S, CHP), jnp.float32),
            pltpu.SemaphoreType.DMA,
        ],
    )(shifted, posx)


def kernel(input, mask, shift_sz, stride, triple_w, flag):
    bz, c, h, w = input.shape
    ch = c // 2
    f2d, l2d = _tc_pre(input[0].reshape(c, N))
    flag = flag.astype(jnp.int32)
    is_m = flag == 1
    mcum = jnp.cumsum(is_m.astype(jnp.int32))
    ucum = jnp.cumsum(1 - is_m.astype(jnp.int32))
    p_cnt = mcum[-1]
    q_cnt = N - p_cnt

    iota = jnp.arange(N, dtype=jnp.int32)
    srcf = jnp.argsort(jnp.where(is_m, iota, iota + N)).astype(
        jnp.int32).reshape(NWORK, JCH, SC)
    srcl = jnp.argsort(jnp.where(is_m, iota + N, iota)).astype(
        jnp.int32).reshape(NWORK, JCH, SC)
    fp, lq = _sc_compact(f2d, l2d, srcf, srcl)
    live = jnp.arange(NPAD, dtype=jnp.int32) < q_cnt
    colneg = jnp.where(live, 0.0, NEG).astype(jnp.float32).reshape(1, NPAD)
    invn = jnp.where(live, jax.lax.rsqrt(jnp.sum(lq * lq, axis=1)),
                     0.0).astype(jnp.float32).reshape(1, NPAD)
    pq = jnp.stack([p_cnt, q_cnt]).astype(jnp.int32)

    grid_spec = pltpu.PrefetchScalarGridSpec(
        num_scalar_prefetch=1,
        grid=(NPAD // BR,),
        in_specs=[
            pl.BlockSpec((BR, CHP), lambda i, pq: (i, 0)),
            pl.BlockSpec((NPAD, CHP), lambda i, pq: (0, 0)),
            pl.BlockSpec((1, NPAD), lambda i, pq: (0, 0)),
            pl.BlockSpec((1, NPAD), lambda i, pq: (0, 0)),
        ],
        out_specs=pl.BlockSpec((BR, CHP), lambda i, pq: (i, 0)),
    )
    shifted = pl.pallas_call(
        _attn_kernel,
        grid_spec=grid_spec,
        out_shape=jax.ShapeDtypeStruct((NPAD, CHP), jnp.float32),
        compiler_params=pltpu.CompilerParams(
            dimension_semantics=("arbitrary",)),
    )(pq, fp, lq, colneg, invn)

    posx = jnp.where(is_m, mcum - 1, N + (iota % (NPAD - N))).reshape(
        NWORK, JCH, SC)
    out2d = _sc_paste(shifted, posx)
    former_masked = _tc_post(out2d).reshape(1, ch, h, w)
    return jnp.concatenate([input, former_masked], axis=1)


# speedup vs baseline: 1.5560x; 1.0514x over previous
import functools

import jax
import jax.numpy as jnp
from jax.experimental import pallas as pl
from jax.experimental.pallas import tpu as pltpu
from jax.experimental.pallas import tpu_sc as plsc

H = 96
W = 96
N = H * W
NPAD = N + 512
CH = 64
CHP = 128
BR = 512
CT = 1024
NEG = -1e30


def _attn_kernel(pq_ref, f_blk, lq_ref, colneg_ref, invn_ref, out_blk):
    i = pl.program_id(0)
    p_cnt = pq_ref[0]
    q_cnt = pq_ref[1]

    @pl.when(i * BR < p_cnt)
    def _compute():
        f = f_blk[...][:, :CH]
        tj = (q_cnt + CT - 1) // CT

        def body(j, carry):
            s, acc = carry
            lt = lq_ref[pl.ds(j * CT, CT), :][:, :CH]
            logits = jax.lax.dot_general(
                f, lt, (((1,), (1,)), ((), ())),
                preferred_element_type=jnp.float32)
            p = jnp.exp(logits * invn_ref[:, pl.ds(j * CT, CT)]
                        + colneg_ref[:, pl.ds(j * CT, CT)])
            s_new = s + jnp.sum(p, axis=1, keepdims=True)
            acc_new = acc + jax.lax.dot_general(
                p, lt, (((1,), (0,)), ((), ())),
                preferred_element_type=jnp.float32)
            return s_new, acc_new

        s0 = jnp.zeros((BR, 1), jnp.float32)
        a0 = jnp.zeros((BR, CH), jnp.float32)
        s, acc = jax.lax.fori_loop(0, tj, body, (s0, a0))
        out_blk[...] = jnp.concatenate(
            [acc / s, jnp.zeros((BR, CHP - CH), jnp.float32)], axis=1)

    @pl.when(i * BR >= p_cnt)
    def _zero():
        out_blk[...] = jnp.zeros((BR, CHP), jnp.float32)


TB = 512


def _pre_kernel(x_blk, f_out, l_out):
    xt = jnp.transpose(x_blk[...])
    z = jnp.zeros((TB, CHP - CH), jnp.float32)
    f_out[...] = jnp.concatenate([xt[:, :CH], z], axis=1)
    l_out[...] = jnp.concatenate([xt[:, CH:], z], axis=1)


def _tc_pre(x):
    return pl.pallas_call(
        _pre_kernel,
        grid=(N // TB,),
        in_specs=[pl.BlockSpec((CHP, TB), lambda i: (0, i))],
        out_specs=[pl.BlockSpec((TB, CHP), lambda i: (i, 0)),
                   pl.BlockSpec((TB, CHP), lambda i: (i, 0))],
        out_shape=[jax.ShapeDtypeStruct((N, CHP), jnp.float32),
                   jax.ShapeDtypeStruct((N, CHP), jnp.float32)],
        compiler_params=pltpu.CompilerParams(
            dimension_semantics=("arbitrary",)),
    )(x)


def _post_kernel(x_blk, out_blk):
    out_blk[...] = jnp.transpose(x_blk[...][:, :CH])


def _tc_post(x):
    return pl.pallas_call(
        _post_kernel,
        grid=(N // TB,),
        in_specs=[pl.BlockSpec((TB, CHP), lambda i: (i, 0))],
        out_specs=pl.BlockSpec((CH, TB), lambda i: (0, i)),
        out_shape=jax.ShapeDtypeStruct((CH, N), jnp.float32),
        compiler_params=pltpu.CompilerParams(
            dimension_semantics=("arbitrary",)),
    )(x)


NC = 2
NS = 16
NWORK = NC * NS
S = N // NWORK
JCH = 3
SC = S // JCH

@functools.cache
def _sc_mesh():
    return plsc.VectorSubcoreMesh(core_axis_name="c", subcore_axis_name="s")


def _compact_body(f2d, l2d, srcf, srcl, fp, lq, idxf_v, idxl_v, rows_f, rows_l, sem):
    wid = jax.lax.axis_index("s") * NC + jax.lax.axis_index("c")
    base = wid * S
    pltpu.sync_copy(srcf.at[wid], idxf_v)
    pltpu.sync_copy(srcl.at[wid], idxl_v)
    copies = []
    for j in range(JCH):
        copies.append(pltpu.async_copy(
            f2d.at[idxf_v.at[j]], rows_f.at[pl.ds(j * SC, SC)], sem))
        copies.append(pltpu.async_copy(
            l2d.at[idxl_v.at[j]], rows_l.at[pl.ds(j * SC, SC)], sem))
    for cp in copies:
        cp.wait()
    pltpu.sync_copy(rows_f, fp.at[pl.ds(base, S)])
    pltpu.sync_copy(rows_l, lq.at[pl.ds(base, S)])


def _sc_compact(f2d, l2d, srcf, srcl):
    return pl.kernel(
        _compact_body,
        out_type=(jax.ShapeDtypeStruct((NPAD, CHP), jnp.float32),
                  jax.ShapeDtypeStruct((NPAD, CHP), jnp.float32)),
        mesh=_sc_mesh(),
        scratch_types=[
            pltpu.VMEM((JCH, SC), jnp.int32),
            pltpu.VMEM((JCH, SC), jnp.int32),
            pltpu.VMEM((S, CHP), jnp.float32),
            pltpu.VMEM((S, CHP), jnp.float32),
            pltpu.SemaphoreType.DMA,
        ],
    )(f2d, l2d, srcf, srcl)


def _paste_body(shifted, posx, out, idx_v, rows_v, sem):
    wid = jax.lax.axis_index("s") * NC + jax.lax.axis_index("c")
    base = wid * S
    pltpu.sync_copy(posx.at[wid], idx_v)
    copies = [pltpu.async_copy(shifted.at[idx_v.at[j]],
                               rows_v.at[pl.ds(j * SC, SC)], sem)
              for j in range(JCH)]
    for cp in copies:
        cp.wait()
    pltpu.sync_copy(rows_v, out.at[pl.ds(base, S)])


def _sc_paste(shifted, posx):
    return pl.kernel(
        _paste_body,
        out_type=jax.ShapeDtypeStruct((N, CHP), jnp.float32),
        mesh=_sc_mesh(),
        scratch_types=[
            pltpu.VMEM((JCH, SC), jnp.int32),
            pltpu.VMEM((S, CHP), jnp.float32),
            pltpu.SemaphoreType.DMA,
        ],
    )(shifted, posx)


def kernel(input, mask, shift_sz, stride, triple_w, flag):
    bz, c, h, w = input.shape
    ch = c // 2
    f2d, l2d = _tc_pre(input[0].reshape(c, N))
    flag = flag.astype(jnp.int32)
    is_m = flag == 1
    mcum = jnp.cumsum(is_m.astype(jnp.int32))
    ucum = jnp.cumsum(1 - is_m.astype(jnp.int32))
    p_cnt = mcum[-1]
    q_cnt = N - p_cnt

    iota = jnp.arange(N, dtype=jnp.int32)
    srcf_flat = jnp.argsort(jnp.where(is_m, iota, iota + N)).astype(jnp.int32)
    srcf = srcf_flat.reshape(NWORK, JCH, SC)
    srcl = jnp.roll(srcf_flat, -p_cnt).reshape(NWORK, JCH, SC)
    fp, lq = _sc_compact(f2d, l2d, srcf, srcl)
    live = jnp.arange(NPAD, dtype=jnp.int32) < q_cnt
    colneg = jnp.where(live, 0.0, NEG).astype(jnp.float32).reshape(1, NPAD)
    invn = jnp.where(live, jax.lax.rsqrt(jnp.sum(lq * lq, axis=1)),
                     0.0).astype(jnp.float32).reshape(1, NPAD)
    pq = jnp.stack([p_cnt, q_cnt]).astype(jnp.int32)

    grid_spec = pltpu.PrefetchScalarGridSpec(
        num_scalar_prefetch=1,
        grid=(NPAD // BR,),
        in_specs=[
            pl.BlockSpec((BR, CHP), lambda i, pq: (i, 0)),
            pl.BlockSpec((NPAD, CHP), lambda i, pq: (0, 0)),
            pl.BlockSpec((1, NPAD), lambda i, pq: (0, 0)),
            pl.BlockSpec((1, NPAD), lambda i, pq: (0, 0)),
        ],
        out_specs=pl.BlockSpec((BR, CHP), lambda i, pq: (i, 0)),
    )
    shifted = pl.pallas_call(
        _attn_kernel,
        grid_spec=grid_spec,
        out_shape=jax.ShapeDtypeStruct((NPAD, CHP), jnp.float32),
        compiler_params=pltpu.CompilerParams(
            dimension_semantics=("arbitrary",)),
    )(pq, fp, lq, colneg, invn)

    posx = jnp.where(is_m, mcum - 1, N + (iota % (NPAD - N))).reshape(
        NWORK, JCH, SC)
    out2d = _sc_paste(shifted, posx)
    former_masked = _tc_post(out2d).reshape(1, ch, h, w)
    return jnp.concatenate([input, former_masked], axis=1)
